# baseline (speedup 1.0000x reference)
import functools

import jax
import jax.numpy as jnp
from jax import lax
from jax.experimental import pallas as pl
from jax.experimental.pallas import tpu as pltpu
from jax.experimental.pallas import tpu_sc as plsc

_N = 10000
_E = 320000
_D = 128
_HD = _D // 2
_ED = 16
_NSUB = 16
_EPS = _E // _NSUB
_C = 80
_NCHUNK = _EPS // _C
_NPR = 5120


def _make_edge_pass(npass, nrows, nreal):
    rpt = nrows // _NSUB

    def body(h_hbm, src_hbm, dst_hbm, e_hbm, out_hbm,
             src_v, dst_v, rows_v, e_v, msg_v, row_v, par_v, agg_s, sem):
        cid = lax.axis_index("c")
        sid = lax.axis_index("s")
        col0 = cid * _HD
        zvec = jnp.zeros((16,), jnp.float32)

        pltpu.sync_copy(src_hbm.at[sid], src_v)
        pltpu.sync_copy(dst_hbm.at[sid], dst_v)

        def zero_body(t, c0):
            for q in range(_D // 16):
                msg_v[t, pl.ds(q * 16, 16)] = zvec
            return c0

        lax.fori_loop(0, _C, zero_body, 0)

        for p in range(npass):
            for t in range(rpt // _C):
                pltpu.sync_copy(msg_v,
                                agg_s.at[pl.ds(sid * rpt + t * _C, _C)])
            plsc.subcore_barrier()

            def chunk_body(j, carry):
                ebase = (cid * (_E // 2) + sid * (_EPS // 2)
                         + j * (_C // 2))
                pltpu.sync_copy(e_hbm.at[pl.ds(ebase, _C // 2)], e_v)
                pltpu.async_copy(h_hbm.at[src_v.at[j]], rows_v, sem).wait()

                for k in range(_C // 16):
                    dvec = dst_v[j, pl.ds(k * 16, 16)]
                    r = lax.shift_right_logical(dvec, 1)
                    if npass > 1:
                        r = r - p * nreal
                        oob = (r < 0) | (r >= nreal)
                        r = jnp.where(oob, nreal, r)
                    row_v[pl.ds(k * 16, 16)] = r
                    par_v[pl.ds(k * 16, 16)] = lax.shift_left(dvec & 1, 6)

                def pair_body(q_r, c2):
                    for s in range(2):
                        i = 2 * q_r + s
                        pp = par_v[pl.ds(i, 16)][0]
                        for q in range(_HD // 16):
                            a = rows_v[i, pl.ds(col0 + q * 16, 16)]
                            b = e_v[q_r, pl.ds(s * _HD + q * 16, 16)]
                            msg_v[i, pl.ds(pp + q * 16, 16)] = (
                                jnp.maximum(a + b, 0.0))
                            msg_v[i, pl.ds((_HD - pp) + q * 16, 16)] = zvec
                    return c2

                lax.fori_loop(0, _C // 2, pair_body, 0)
                pltpu.sync_copy(msg_v, agg_s.at[row_v], add=True)
                return carry

            lax.fori_loop(0, _NCHUNK, chunk_body, 0)
            plsc.subcore_barrier()
            obase = (cid * npass + p) * nrows + sid * rpt
            pltpu.sync_copy(agg_s.at[pl.ds(sid * rpt, rpt)],
                            out_hbm.at[pl.ds(obase, rpt)])
            if npass > 1 and p < npass - 1:
                def rezero(t, c0):
                    for q in range(_D // 16):
                        msg_v[t, pl.ds(q * 16, 16)] = zvec
                    return c0

                lax.fori_loop(0, _C, rezero, 0)

    return functools.partial(
        pl.kernel,
        out_type=jax.ShapeDtypeStruct((npass * 2 * nrows, _D), jnp.float32),
        mesh=plsc.VectorSubcoreMesh(core_axis_name="c",
                                    subcore_axis_name="s"),
        scratch_types=[
            pltpu.VMEM((_NCHUNK, _C), jnp.int32),
            pltpu.VMEM((_NCHUNK, _C), jnp.int32),
            pltpu.VMEM((_C, _D), jnp.float32),
            pltpu.VMEM((_C // 2, _D), jnp.float32),
            pltpu.VMEM((_C, _D), jnp.float32),
            pltpu.VMEM((_C,), jnp.int32),
            pltpu.VMEM((_C + 16,), jnp.int32),
            pltpu.VMEM_SHARED((nrows, _D), jnp.float32),
            pltpu.SemaphoreType.DMA,
        ],
    )(body)


_edge_pass_two = _make_edge_pass(2, 2560, 2500)


def _edge_mlp_body(attr_ref, w_ref, b_ref, out_ref):
    out_ref[...] = (
        jnp.dot(attr_ref[...], w_ref[0], preferred_element_type=jnp.float32)
        + b_ref[0]
    )


def _edge_mlp(attr2, w_blkdiag, b_pair):
    be = 8000
    return pl.pallas_call(
        _edge_mlp_body,
        grid=(2, _E // be),
        in_specs=[
            pl.BlockSpec((be // 2, 2 * _ED), lambda c, i: (i, 0)),
            pl.BlockSpec((1, 2 * _ED, _D), lambda c, i: (c, 0, 0)),
            pl.BlockSpec((1, 1, _D), lambda c, i: (c, 0, 0)),
        ],
        out_specs=pl.BlockSpec((be // 2, _D),
                               lambda c, i: (c * (_E // be) + i, 0)),
        out_shape=jax.ShapeDtypeStruct((_E, _D), jnp.float32),
    )(attr2, w_blkdiag, b_pair)


def _lrelu(x):
    return jnp.where(x > 0, x, 0.01 * x)


def _node_body(h_ref, p0_ref, p1_ref, w_ref, b_ref, g_ref, be_ref, eps_ref,
               out_ref):
    h = h_ref[...]
    agg = jnp.concatenate([p0_ref[...], p1_ref[...]], axis=1)
    a = (1.0 + eps_ref[0]) * h + agg
    z = jnp.dot(a, w_ref[...], preferred_element_type=jnp.float32) + b_ref[...]
    mean = jnp.mean(z, axis=0, keepdims=True)
    var = jnp.mean((z - mean) ** 2, axis=0, keepdims=True)
    zn = (z - mean) / jnp.sqrt(var + 1e-5) * g_ref[...] + be_ref[...]
    out_ref[...] = _lrelu(_lrelu(zn))


def _node_update(h, p0, p1, w, b, g, be, eps):
    return pl.pallas_call(
        _node_body,
        out_shape=jax.ShapeDtypeStruct((_N, _D), jnp.float32),
    )(h, p0, p1, w, b, g, be, eps)


def _cls_body(h_ref, w1_ref, b1_ref, w_ref, b_ref, fin_ref, finb_ref, out_ref):
    c = (jnp.dot(h_ref[...], w1_ref[...], preferred_element_type=jnp.float32)
         + b1_ref[...])
    for i in range(2):
        c = (jnp.dot(c, w_ref[i], preferred_element_type=jnp.float32)
             + b_ref[i])
        c = _lrelu(c)
    logits = (jnp.dot(c, fin_ref[...], preferred_element_type=jnp.float32)
              + finb_ref[0])
    out_ref[...] = jax.nn.sigmoid(logits)


def _classifier(h, w1, b1, w, b, finw_pad, finb):
    return pl.pallas_call(
        _cls_body,
        out_shape=jax.ShapeDtypeStruct((_N, _D), jnp.float32),
    )(h, w1, b1.reshape(1, -1), w, b.reshape(b.shape[0], 1, -1), finw_pad,
      finb)


def kernel(x, edge_index, edge_attr, eps1, eW1, eb1, W1, b1, g1, be1,
           conv_eps, conv_eW, conv_eb, conv_W, conv_b, conv_g, conv_be,
           clsW1, clsb1, clsW, clsb, finW, finb):
    src = edge_index[0].astype(jnp.int32).reshape(_NSUB, _NCHUNK, _C)
    dst = edge_index[1].astype(jnp.int32).reshape(_NSUB, _NCHUNK, _C)
    attr2 = edge_attr.reshape(_E // 2, 2 * _ED)
    zpad = jnp.zeros((_ED, _HD), jnp.float32)
    finw_pad = jnp.pad(finW, ((0, 0), (0, _D - finW.shape[1])))

    layers = [(eps1.reshape(1), eW1, eb1, W1, b1, g1, be1)]
    for i in range(conv_eps.shape[0]):
        layers.append((conv_eps[i].reshape(1), conv_eW[i], conv_eb[i],
                       conv_W[i], conv_b[i], conv_g[i], conv_be[i]))

    h = x
    for li, (eps, ew, eb, w, b, g, be) in enumerate(layers):
        w_blkdiag = jnp.stack([
            jnp.block([[ew[:, :_HD], zpad], [zpad, ew[:, :_HD]]]),
            jnp.block([[ew[:, _HD:], zpad], [zpad, ew[:, _HD:]]]),
        ])
        b_pair = jnp.stack([
            jnp.concatenate([eb[:_HD], eb[:_HD]]).reshape(1, _D),
            jnp.concatenate([eb[_HD:], eb[_HD:]]).reshape(1, _D),
        ])
        e = _edge_mlp(attr2, w_blkdiag, b_pair)
        parts = _edge_pass_two(h, src, dst, e)
        p0 = jnp.concatenate([parts[0:2500], parts[2560:5060]]
                             ).reshape(_N, _HD)
        p1 = jnp.concatenate([parts[5120:7620], parts[7680:10180]]
                             ).reshape(_N, _HD)
        h = _node_update(h, p0, p1, w, b.reshape(1, _D), g.reshape(1, _D),
                         be.reshape(1, _D), eps)

    out = _classifier(h, clsW1, clsb1, clsW, clsb, finw_pad, finb)
    return out[:, 0:1]


# speedup vs baseline: 1.1485x; 1.1485x over previous
import functools

import jax
import jax.numpy as jnp
from jax import lax
from jax.experimental import pallas as pl
from jax.experimental.pallas import tpu as pltpu
from jax.experimental.pallas import tpu_sc as plsc

_N = 10000
_E = 320000
_D = 128
_HD = _D // 2
_ED = 16
_NSUB = 16
_EPS = _E // _NSUB
_C = 80
_NCHUNK = _EPS // _C
_NPR = 5120


def _make_edge_pass(npass, nrows, nreal):
    rpt = nrows // _NSUB

    def body(h_hbm, src_hbm, dst_hbm, e_hbm, out_hbm,
             src_v, dst_v, rows_v, e_v, msg_v, row_v, par_v, agg_s, sem):
        cid = lax.axis_index("c")
        sid = lax.axis_index("s")
        col0 = cid * _HD
        zvec = jnp.zeros((16,), jnp.float32)

        pltpu.sync_copy(src_hbm.at[sid], src_v)
        pltpu.sync_copy(dst_hbm.at[sid], dst_v)

        def zero_body(t, c0):
            for q in range(_D // 16):
                msg_v[t, pl.ds(q * 16, 16)] = zvec
            return c0

        lax.fori_loop(0, _C, zero_body, 0)

        for p in range(npass):
            for t in range(rpt // _C):
                pltpu.sync_copy(msg_v,
                                agg_s.at[pl.ds(sid * rpt + t * _C, _C)])
            plsc.subcore_barrier()

            def chunk_body(j, carry):
                ebase = (cid * (_E // 2) + sid * (_EPS // 2)
                         + j * (_C // 2))
                gcopy = pltpu.async_copy(h_hbm.at[src_v.at[j]], rows_v,
                                         sem)
                pltpu.sync_copy(e_hbm.at[pl.ds(ebase, _C // 2)], e_v)
                gcopy.wait()

                for k in range(_C // 16):
                    dvec = dst_v[j, pl.ds(k * 16, 16)]
                    r = lax.shift_right_logical(dvec, 1)
                    if npass > 1:
                        r = r - p * nreal
                        oob = (r < 0) | (r >= nreal)
                        r = jnp.where(oob, nreal, r)
                    row_v[pl.ds(k * 16, 16)] = r
                    par_v[pl.ds(k * 16, 16)] = lax.shift_left(dvec & 1, 6)

                def pair_body(q_r, c2):
                    for s in range(2):
                        i = 2 * q_r + s
                        pp = par_v[pl.ds(i, 16)][0]
                        for q in range(_HD // 16):
                            a = rows_v[i, pl.ds(col0 + q * 16, 16)]
                            b = e_v[q_r, pl.ds(s * _HD + q * 16, 16)]
                            msg_v[i, pl.ds(pp + q * 16, 16)] = (
                                jnp.maximum(a + b, 0.0))
                            msg_v[i, pl.ds((_HD - pp) + q * 16, 16)] = zvec
                    return c2

                lax.fori_loop(0, _C // 2, pair_body, 0)
                pltpu.sync_copy(msg_v, agg_s.at[row_v], add=True)
                return carry

            lax.fori_loop(0, _NCHUNK, chunk_body, 0)
            plsc.subcore_barrier()
            obase = (cid * npass + p) * nrows + sid * rpt
            pltpu.sync_copy(agg_s.at[pl.ds(sid * rpt, rpt)],
                            out_hbm.at[pl.ds(obase, rpt)])
            if npass > 1 and p < npass - 1:
                def rezero(t, c0):
                    for q in range(_D // 16):
                        msg_v[t, pl.ds(q * 16, 16)] = zvec
                    return c0

                lax.fori_loop(0, _C, rezero, 0)

    return functools.partial(
        pl.kernel,
        out_type=jax.ShapeDtypeStruct((npass * 2 * nrows, _D), jnp.float32),
        mesh=plsc.VectorSubcoreMesh(core_axis_name="c",
                                    subcore_axis_name="s"),
        scratch_types=[
            pltpu.VMEM((_NCHUNK, _C), jnp.int32),
            pltpu.VMEM((_NCHUNK, _C), jnp.int32),
            pltpu.VMEM((_C, _D), jnp.float32),
            pltpu.VMEM((_C // 2, _D), jnp.float32),
            pltpu.VMEM((_C, _D), jnp.float32),
            pltpu.VMEM((_C,), jnp.int32),
            pltpu.VMEM((_C + 16,), jnp.int32),
            pltpu.VMEM_SHARED((nrows, _D), jnp.float32),
            pltpu.SemaphoreType.DMA,
        ],
    )(body)


_edge_pass_two = _make_edge_pass(2, 2560, 2500)


def _edge_mlp_body(attr_ref, w_ref, b_ref, out_ref):
    out_ref[...] = (
        jnp.dot(attr_ref[...], w_ref[0], preferred_element_type=jnp.float32)
        + b_ref[0]
    )


def _edge_mlp(attr2, w_blkdiag, b_pair):
    be = 8000
    return pl.pallas_call(
        _edge_mlp_body,
        grid=(2, _E // be),
        in_specs=[
            pl.BlockSpec((be // 2, 2 * _ED), lambda c, i: (i, 0)),
            pl.BlockSpec((1, 2 * _ED, _D), lambda c, i: (c, 0, 0)),
            pl.BlockSpec((1, 1, _D), lambda c, i: (c, 0, 0)),
        ],
        out_specs=pl.BlockSpec((be // 2, _D),
                               lambda c, i: (c * (_E // be) + i, 0)),
        out_shape=jax.ShapeDtypeStruct((_E, _D), jnp.float32),
    )(attr2, w_blkdiag, b_pair)


def _lrelu(x):
    return jnp.where(x > 0, x, 0.01 * x)


def _node_body(h_ref, p0_ref, p1_ref, w_ref, b_ref, g_ref, be_ref, eps_ref,
               out_ref):
    h = h_ref[...]
    agg = jnp.concatenate([p0_ref[...], p1_ref[...]], axis=1)
    a = (1.0 + eps_ref[0]) * h + agg
    z = jnp.dot(a, w_ref[...], preferred_element_type=jnp.float32) + b_ref[...]
    mean = jnp.mean(z, axis=0, keepdims=True)
    var = jnp.mean((z - mean) ** 2, axis=0, keepdims=True)
    zn = (z - mean) / jnp.sqrt(var + 1e-5) * g_ref[...] + be_ref[...]
    out_ref[...] = _lrelu(_lrelu(zn))


def _node_update(h, p0, p1, w, b, g, be, eps):
    return pl.pallas_call(
        _node_body,
        out_shape=jax.ShapeDtypeStruct((_N, _D), jnp.float32),
    )(h, p0, p1, w, b, g, be, eps)


def _cls_body(h_ref, w1_ref, b1_ref, w_ref, b_ref, fin_ref, finb_ref, out_ref):
    c = (jnp.dot(h_ref[...], w1_ref[...], preferred_element_type=jnp.float32)
         + b1_ref[...])
    for i in range(2):
        c = (jnp.dot(c, w_ref[i], preferred_element_type=jnp.float32)
             + b_ref[i])
        c = _lrelu(c)
    logits = (jnp.dot(c, fin_ref[...], preferred_element_type=jnp.float32)
              + finb_ref[0])
    out_ref[...] = jax.nn.sigmoid(logits)


def _classifier(h, w1, b1, w, b, finw_pad, finb):
    return pl.pallas_call(
        _cls_body,
        out_shape=jax.ShapeDtypeStruct((_N, _D), jnp.float32),
    )(h, w1, b1.reshape(1, -1), w, b.reshape(b.shape[0], 1, -1), finw_pad,
      finb)


def kernel(x, edge_index, edge_attr, eps1, eW1, eb1, W1, b1, g1, be1,
           conv_eps, conv_eW, conv_eb, conv_W, conv_b, conv_g, conv_be,
           clsW1, clsb1, clsW, clsb, finW, finb):
    src = edge_index[0].astype(jnp.int32).reshape(_NSUB, _NCHUNK, _C)
    dst = edge_index[1].astype(jnp.int32).reshape(_NSUB, _NCHUNK, _C)
    attr2 = edge_attr.reshape(_E // 2, 2 * _ED)
    zpad = jnp.zeros((_ED, _HD), jnp.float32)
    finw_pad = jnp.pad(finW, ((0, 0), (0, _D - finW.shape[1])))

    layers = [(eps1.reshape(1), eW1, eb1, W1, b1, g1, be1)]
    for i in range(conv_eps.shape[0]):
        layers.append((conv_eps[i].reshape(1), conv_eW[i], conv_eb[i],
                       conv_W[i], conv_b[i], conv_g[i], conv_be[i]))

    h = x
    for li, (eps, ew, eb, w, b, g, be) in enumerate(layers):
        w_blkdiag = jnp.stack([
            jnp.block([[ew[:, :_HD], zpad], [zpad, ew[:, :_HD]]]),
            jnp.block([[ew[:, _HD:], zpad], [zpad, ew[:, _HD:]]]),
        ])
        b_pair = jnp.stack([
            jnp.concatenate([eb[:_HD], eb[:_HD]]).reshape(1, _D),
            jnp.concatenate([eb[_HD:], eb[_HD:]]).reshape(1, _D),
        ])
        e = _edge_mlp(attr2, w_blkdiag, b_pair)
        parts = _edge_pass_two(h, src, dst, e)
        p0 = jnp.concatenate([parts[0:2500], parts[2560:5060]]
                             ).reshape(_N, _HD)
        p1 = jnp.concatenate([parts[5120:7620], parts[7680:10180]]
                             ).reshape(_N, _HD)
        h = _node_update(h, p0, p1, w, b.reshape(1, _D), g.reshape(1, _D),
                         be.reshape(1, _D), eps)

    out = _classifier(h, clsW1, clsb1, clsW, clsb, finw_pad, finb)
    return out[:, 0:1]


# speedup vs baseline: 1.5202x; 1.3236x over previous
import functools

import jax
import jax.numpy as jnp
from jax import lax
from jax.experimental import pallas as pl
from jax.experimental.pallas import tpu as pltpu
from jax.experimental.pallas import tpu_sc as plsc

_N = 10000
_E = 320000
_D = 128
_HD = _D // 2
_ED = 16
_NSUB = 16
_EPS = _E // _NSUB
_C = 80
_NCHUNK = _EPS // _C
_NPR = 5120


def _make_edge_pass(npass, nrows, nreal):
    rpt = nrows // _NSUB

    def body(h_hbm, src_hbm, dst_hbm, e_hbm, out_hbm,
             src_v, dst_v, rows_v, e_v, msg_v, row_v, par_v, agg_s,
             gsem0, gsem1, esem0, esem1):
        cid = lax.axis_index("c")
        sid = lax.axis_index("s")
        col0 = cid * _HD
        zvec = jnp.zeros((16,), jnp.float32)
        gsems = (gsem0, gsem1)
        esems = (esem0, esem1)
        ebase0 = cid * (_E // 2) + sid * (_EPS // 2)

        pltpu.sync_copy(src_hbm.at[sid], src_v)
        pltpu.sync_copy(dst_hbm.at[sid], dst_v)

        def zero_body(t, c0):
            for q in range(_D // 16):
                msg_v[t, pl.ds(q * 16, 16)] = zvec
            return c0

        lax.fori_loop(0, _C, zero_body, 0)

        def issue_loads(j, buf):
            pltpu.async_copy(h_hbm.at[src_v.at[j]], rows_v.at[buf],
                             gsems[buf])
            pltpu.async_copy(e_hbm.at[pl.ds(ebase0 + j * (_C // 2),
                                            _C // 2)],
                             e_v.at[buf], esems[buf])

        def wait_loads(j, buf):
            pltpu.make_async_copy(h_hbm.at[src_v.at[j]], rows_v.at[buf],
                                  gsems[buf]).wait()
            pltpu.make_async_copy(e_hbm.at[pl.ds(ebase0 + j * (_C // 2),
                                                 _C // 2)],
                                  e_v.at[buf], esems[buf]).wait()

        for p in range(npass):
            for t in range(rpt // _C):
                pltpu.sync_copy(msg_v,
                                agg_s.at[pl.ds(sid * rpt + t * _C, _C)])
            plsc.subcore_barrier()

            issue_loads(0, 0)

            def chunk2_body(j2, carry):
                for buf in range(2):
                    j = 2 * j2 + buf

                    @pl.when(j < _NCHUNK - 1)
                    def _prefetch():
                        issue_loads(j + 1, 1 - buf)

                    wait_loads(j, buf)

                    for k in range(_C // 16):
                        dvec = dst_v[j, pl.ds(k * 16, 16)]
                        r = lax.shift_right_logical(dvec, 1)
                        if npass > 1:
                            r = r - p * nreal
                            oob = (r < 0) | (r >= nreal)
                            r = jnp.where(oob, nreal, r)
                        row_v[pl.ds(k * 16, 16)] = r
                        par_v[pl.ds(k * 16, 16)] = lax.shift_left(
                            dvec & 1, 6)

                    def pair_body(q_r, c2):
                        for s in range(2):
                            i = 2 * q_r + s
                            pp = par_v[pl.ds(i, 16)][0]
                            for q in range(_HD // 16):
                                a = rows_v[buf, i,
                                           pl.ds(col0 + q * 16, 16)]
                                b = e_v[buf, q_r,
                                        pl.ds(s * _HD + q * 16, 16)]
                                msg_v[i, pl.ds(pp + q * 16, 16)] = (
                                    jnp.maximum(a + b, 0.0))
                                msg_v[i, pl.ds((_HD - pp) + q * 16,
                                               16)] = zvec
                        return c2

                    lax.fori_loop(0, _C // 2, pair_body, 0)
                    pltpu.sync_copy(msg_v, agg_s.at[row_v], add=True)
                return carry

            lax.fori_loop(0, _NCHUNK // 2, chunk2_body, 0)
            plsc.subcore_barrier()
            obase = (cid * npass + p) * nrows + sid * rpt
            pltpu.sync_copy(agg_s.at[pl.ds(sid * rpt, rpt)],
                            out_hbm.at[pl.ds(obase, rpt)])
            if npass > 1 and p < npass - 1:
                def rezero(t, c0):
                    for q in range(_D // 16):
                        msg_v[t, pl.ds(q * 16, 16)] = zvec
                    return c0

                lax.fori_loop(0, _C, rezero, 0)

    return functools.partial(
        pl.kernel,
        out_type=jax.ShapeDtypeStruct((npass * 2 * nrows, _D), jnp.float32),
        mesh=plsc.VectorSubcoreMesh(core_axis_name="c",
                                    subcore_axis_name="s"),
        scratch_types=[
            pltpu.VMEM((_NCHUNK, _C), jnp.int32),
            pltpu.VMEM((_NCHUNK, _C), jnp.int32),
            pltpu.VMEM((2, _C, _D), jnp.float32),
            pltpu.VMEM((2, _C // 2, _D), jnp.float32),
            pltpu.VMEM((_C, _D), jnp.float32),
            pltpu.VMEM((_C,), jnp.int32),
            pltpu.VMEM((_C + 16,), jnp.int32),
            pltpu.VMEM_SHARED((nrows, _D), jnp.float32),
            pltpu.SemaphoreType.DMA,
            pltpu.SemaphoreType.DMA,
            pltpu.SemaphoreType.DMA,
            pltpu.SemaphoreType.DMA,
        ],
    )(body)


_edge_pass_two = _make_edge_pass(2, 2560, 2500)


def _edge_mlp_body(attr_ref, w_ref, b_ref, out_ref):
    out_ref[...] = (
        jnp.dot(attr_ref[...], w_ref[0], preferred_element_type=jnp.float32)
        + b_ref[0]
    )


def _edge_mlp(attr2, w_blkdiag, b_pair):
    be = 8000
    return pl.pallas_call(
        _edge_mlp_body,
        grid=(2, _E // be),
        in_specs=[
            pl.BlockSpec((be // 2, 2 * _ED), lambda c, i: (i, 0)),
            pl.BlockSpec((1, 2 * _ED, _D), lambda c, i: (c, 0, 0)),
            pl.BlockSpec((1, 1, _D), lambda c, i: (c, 0, 0)),
        ],
        out_specs=pl.BlockSpec((be // 2, _D),
                               lambda c, i: (c * (_E // be) + i, 0)),
        out_shape=jax.ShapeDtypeStruct((_E, _D), jnp.float32),
    )(attr2, w_blkdiag, b_pair)


def _lrelu(x):
    return jnp.where(x > 0, x, 0.01 * x)


def _node_body(h_ref, p0_ref, p1_ref, w_ref, b_ref, g_ref, be_ref, eps_ref,
               out_ref):
    h = h_ref[...]
    agg = jnp.concatenate([p0_ref[...], p1_ref[...]], axis=1)
    a = (1.0 + eps_ref[0]) * h + agg
    z = jnp.dot(a, w_ref[...], preferred_element_type=jnp.float32) + b_ref[...]
    mean = jnp.mean(z, axis=0, keepdims=True)
    var = jnp.mean((z - mean) ** 2, axis=0, keepdims=True)
    zn = (z - mean) / jnp.sqrt(var + 1e-5) * g_ref[...] + be_ref[...]
    out_ref[...] = _lrelu(_lrelu(zn))


def _node_update(h, p0, p1, w, b, g, be, eps):
    return pl.pallas_call(
        _node_body,
        out_shape=jax.ShapeDtypeStruct((_N, _D), jnp.float32),
    )(h, p0, p1, w, b, g, be, eps)


def _cls_body(h_ref, w1_ref, b1_ref, w_ref, b_ref, fin_ref, finb_ref, out_ref):
    c = (jnp.dot(h_ref[...], w1_ref[...], preferred_element_type=jnp.float32)
         + b1_ref[...])
    for i in range(2):
        c = (jnp.dot(c, w_ref[i], preferred_element_type=jnp.float32)
             + b_ref[i])
        c = _lrelu(c)
    logits = (jnp.dot(c, fin_ref[...], preferred_element_type=jnp.float32)
              + finb_ref[0])
    out_ref[...] = jax.nn.sigmoid(logits)


def _classifier(h, w1, b1, w, b, finw_pad, finb):
    return pl.pallas_call(
        _cls_body,
        out_shape=jax.ShapeDtypeStruct((_N, _D), jnp.float32),
    )(h, w1, b1.reshape(1, -1), w, b.reshape(b.shape[0], 1, -1), finw_pad,
      finb)


def kernel(x, edge_index, edge_attr, eps1, eW1, eb1, W1, b1, g1, be1,
           conv_eps, conv_eW, conv_eb, conv_W, conv_b, conv_g, conv_be,
           clsW1, clsb1, clsW, clsb, finW, finb):
    src = edge_index[0].astype(jnp.int32).reshape(_NSUB, _NCHUNK, _C)
    dst = edge_index[1].astype(jnp.int32).reshape(_NSUB, _NCHUNK, _C)
    attr2 = edge_attr.reshape(_E // 2, 2 * _ED)
    zpad = jnp.zeros((_ED, _HD), jnp.float32)
    finw_pad = jnp.pad(finW, ((0, 0), (0, _D - finW.shape[1])))

    layers = [(eps1.reshape(1), eW1, eb1, W1, b1, g1, be1)]
    for i in range(conv_eps.shape[0]):
        layers.append((conv_eps[i].reshape(1), conv_eW[i], conv_eb[i],
                       conv_W[i], conv_b[i], conv_g[i], conv_be[i]))

    h = x
    for li, (eps, ew, eb, w, b, g, be) in enumerate(layers):
        w_blkdiag = jnp.stack([
            jnp.block([[ew[:, :_HD], zpad], [zpad, ew[:, :_HD]]]),
            jnp.block([[ew[:, _HD:], zpad], [zpad, ew[:, _HD:]]]),
        ])
        b_pair = jnp.stack([
            jnp.concatenate([eb[:_HD], eb[:_HD]]).reshape(1, _D),
            jnp.concatenate([eb[_HD:], eb[_HD:]]).reshape(1, _D),
        ])
        e = _edge_mlp(attr2, w_blkdiag, b_pair)
        parts = _edge_pass_two(h, src, dst, e)
        p0 = jnp.concatenate([parts[0:2500], parts[2560:5060]]
                             ).reshape(_N, _HD)
        p1 = jnp.concatenate([parts[5120:7620], parts[7680:10180]]
                             ).reshape(_N, _HD)
        h = _node_update(h, p0, p1, w, b.reshape(1, _D), g.reshape(1, _D),
                         be.reshape(1, _D), eps)

    out = _classifier(h, clsW1, clsb1, clsW, clsb, finw_pad, finb)
    return out[:, 0:1]
